# baseline (speedup 1.0000x reference)
import jax
import jax.numpy as jnp
from jax.experimental import pallas as pl
from jax.experimental.pallas import tpu as pltpu

TOKENS = 16384
D_MODEL = 2048
NUM_EXPERTS = 16
BLOCK_T = 512
NEG_INF = float("-inf")


def _gating_kernel(x_ref, gw_ref, b_ref, w_ref, idx_ref, topw_ref):
    s = jax.lax.dot_general(
        x_ref[...], gw_ref[...], (((1,), (1,)), ((), ())),
        preferred_element_type=jnp.float32)
    s = s + b_ref[...]
    st = s.T
    sub = jax.lax.broadcasted_iota(jnp.int32, st.shape, 0)

    m0 = jnp.max(st, axis=0, keepdims=True)
    i0 = jnp.min(jnp.where(st == m0, sub, NUM_EXPERTS), axis=0, keepdims=True)
    masked = jnp.where(sub == i0, NEG_INF, st)
    m1 = jnp.max(masked, axis=0, keepdims=True)
    i1 = jnp.min(jnp.where(masked == m1, sub, NUM_EXPERTS), axis=0,
                 keepdims=True)

    e = jnp.exp(m1 - m0)
    w0 = 1.0 / (1.0 + e)
    w1 = e * w0

    w_ref[...] = jnp.where(sub == i0, w0, jnp.where(sub == i1, w1, 0.0))
    idx_ref[...] = jnp.concatenate([i0, i1], axis=0)
    topw_ref[...] = jnp.concatenate([w0, w1], axis=0)


def kernel(x, gate_w, gate_b):
    b2 = gate_b.reshape(1, NUM_EXPERTS)
    grid = (TOKENS // BLOCK_T,)
    weights_t, idx_t, topw_t = pl.pallas_call(
        _gating_kernel,
        grid=grid,
        in_specs=[
            pl.BlockSpec((BLOCK_T, D_MODEL), lambda i: (i, 0)),
            pl.BlockSpec((NUM_EXPERTS, D_MODEL), lambda i: (0, 0)),
            pl.BlockSpec((1, NUM_EXPERTS), lambda i: (0, 0)),
        ],
        out_specs=[
            pl.BlockSpec((NUM_EXPERTS, BLOCK_T), lambda i: (0, i)),
            pl.BlockSpec((2, BLOCK_T), lambda i: (0, i)),
            pl.BlockSpec((2, BLOCK_T), lambda i: (0, i)),
        ],
        out_shape=[
            jax.ShapeDtypeStruct((NUM_EXPERTS, TOKENS), jnp.float32),
            jax.ShapeDtypeStruct((2, TOKENS), jnp.int32),
            jax.ShapeDtypeStruct((2, TOKENS), jnp.float32),
        ],
        compiler_params=pltpu.CompilerParams(
            dimension_semantics=("parallel",)),
    )(x, gate_w, b2)
    return (weights_t.T, idx_t.T, topw_t.T)


# speedup vs baseline: 1.2002x; 1.2002x over previous
import jax
import jax.numpy as jnp
from jax.experimental import pallas as pl
from jax.experimental.pallas import tpu as pltpu

TOKENS = 16384
D_MODEL = 2048
NUM_EXPERTS = 16
BLOCK_T = 1024
NEG_INF = float("-inf")


def _gating_kernel(x_ref, gw_ref, b_ref, w_ref, idx_ref, topw_ref):
    s = jax.lax.dot_general(
        x_ref[...], gw_ref[...], (((1,), (1,)), ((), ())),
        preferred_element_type=jnp.float32)
    s = s + b_ref[...]
    st = s.T
    sub = jax.lax.broadcasted_iota(jnp.int32, st.shape, 0)

    m0 = jnp.max(st, axis=0, keepdims=True)
    i0 = jnp.min(jnp.where(st == m0, sub, NUM_EXPERTS), axis=0, keepdims=True)
    masked = jnp.where(sub == i0, NEG_INF, st)
    m1 = jnp.max(masked, axis=0, keepdims=True)
    i1 = jnp.min(jnp.where(masked == m1, sub, NUM_EXPERTS), axis=0,
                 keepdims=True)

    e = jnp.exp(m1 - m0)
    w0 = 1.0 / (1.0 + e)
    w1 = e * w0

    w_ref[...] = jnp.where(sub == i0, w0, jnp.where(sub == i1, w1, 0.0))
    idx_ref[...] = jnp.concatenate([i0, i1], axis=0)
    topw_ref[...] = jnp.concatenate([w0, w1], axis=0)


def kernel(x, gate_w, gate_b):
    b2 = gate_b.reshape(1, NUM_EXPERTS)
    grid = (TOKENS // BLOCK_T,)
    weights_t, idx_t, topw_t = pl.pallas_call(
        _gating_kernel,
        grid=grid,
        in_specs=[
            pl.BlockSpec((BLOCK_T, D_MODEL), lambda i: (i, 0)),
            pl.BlockSpec((NUM_EXPERTS, D_MODEL), lambda i: (0, 0)),
            pl.BlockSpec((1, NUM_EXPERTS), lambda i: (0, 0)),
        ],
        out_specs=[
            pl.BlockSpec((NUM_EXPERTS, BLOCK_T), lambda i: (0, i)),
            pl.BlockSpec((2, BLOCK_T), lambda i: (0, i)),
            pl.BlockSpec((2, BLOCK_T), lambda i: (0, i)),
        ],
        out_shape=[
            jax.ShapeDtypeStruct((NUM_EXPERTS, TOKENS), jnp.float32),
            jax.ShapeDtypeStruct((2, TOKENS), jnp.int32),
            jax.ShapeDtypeStruct((2, TOKENS), jnp.float32),
        ],
        compiler_params=pltpu.CompilerParams(
            dimension_semantics=("parallel",)),
    )(x, gate_w, b2)
    return (weights_t.T, idx_t.T, topw_t.T)
